# baseline (speedup 1.0000x reference)
import jax
import jax.numpy as jnp
from jax.experimental import pallas as pl
from jax.experimental.pallas import tpu as pltpu

_B = 5000
_N = 20000
_C = 256
_BT = 512
_NT = 2048
_NB = 10
_NN = 10
_BP = _BT * _NB
_NP = _NT * _NN


def _roi_body(boxes_ref, coords_ref, feat_ref, mask_ref, bf_ref,
              sums_ref, counts_ref):
    i_n = pl.program_id(0)
    i_b = pl.program_id(1)

    x = coords_ref[0:1, :]
    y = coords_ref[1:2, :]
    s = coords_ref[2:3, :]
    x0 = boxes_ref[:, 0:1]
    y0 = boxes_ref[:, 1:2]
    x1 = boxes_ref[:, 2:3]
    y1 = boxes_ref[:, 3:4]
    af = boxes_ref[:, 4:5]

    inside = (x0 <= x) & (x < x1) & (y0 <= y) & (y < y1) & (s == af)
    mask_ref[...] = inside

    f = feat_ref[...]
    fh = f.astype(jnp.bfloat16)
    fl = (f - fh.astype(jnp.float32)).astype(jnp.bfloat16)
    m = inside.astype(jnp.bfloat16)
    part = (jnp.dot(m, fh, preferred_element_type=jnp.float32)
            + jnp.dot(m, fl, preferred_element_type=jnp.float32))
    cnt = jnp.sum(inside.astype(jnp.float32), axis=1, keepdims=True)

    rows = pl.ds(i_b * _BT, _BT)

    @pl.when(i_n == 0)
    def _init():
        sums_ref[rows, :] = part
        counts_ref[rows, :] = cnt

    @pl.when(i_n > 0)
    def _acc():
        sums_ref[rows, :] = sums_ref[rows, :] + part
        counts_ref[rows, :] = counts_ref[rows, :] + cnt

    @pl.when(i_n == _NN - 1)
    def _fin():
        tot = sums_ref[rows, :]
        c = jnp.maximum(counts_ref[rows, :], 1.0)
        bf_ref[...] = tot / c


def kernel(coords, features, bbox_tensor, bbox_sample_association):
    af = bbox_sample_association.astype(jnp.float32)
    boxes = jnp.concatenate(
        [bbox_tensor[:, 0, :], bbox_tensor[:, 1, :], af[:, None]], axis=-1)
    boxes_p = jnp.pad(boxes, ((0, _BP - _B), (0, 8 - 5)),
                      constant_values=1e30)
    coords_t = jnp.pad(coords.T, ((0, 8 - 3), (0, _NP - _N)),
                       constant_values=-1e30)
    feat_p = jnp.pad(features, ((0, _NP - _N), (0, 0)))

    grid = (_NN, _NB)
    is_inside, box_features = pl.pallas_call(
        _roi_body,
        grid=grid,
        in_specs=[
            pl.BlockSpec((_BT, 8), lambda i_n, i_b: (i_b, 0)),
            pl.BlockSpec((8, _NT), lambda i_n, i_b: (0, i_n)),
            pl.BlockSpec((_NT, _C), lambda i_n, i_b: (i_n, 0)),
        ],
        out_specs=[
            pl.BlockSpec((_BT, _NT), lambda i_n, i_b: (i_b, i_n)),
            pl.BlockSpec((_BT, _C), lambda i_n, i_b: (i_b, 0)),
        ],
        out_shape=[
            jax.ShapeDtypeStruct((_B, _N), jnp.bool_),
            jax.ShapeDtypeStruct((_B, _C), jnp.float32),
        ],
        scratch_shapes=[
            pltpu.VMEM((_BP, _C), jnp.float32),
            pltpu.VMEM((_BP, 1), jnp.float32),
        ],
    )(boxes_p, coords_t, feat_p)
    return (box_features, is_inside)
